# baseline (speedup 1.0000x reference)
import functools

import jax
import jax.numpy as jnp
from jax import lax
from jax.experimental import pallas as pl
from jax.experimental.pallas import tpu as pltpu
from jax.experimental.pallas import tpu_sc as plsc

_NUM_SC_CORES = 2
_NUM_SC_SUBCORES = 16
_REP = 16


def _g0_body(q0_ref, g0_ref):
    C, R, W = g0_ref.shape
    g0_ref[...] = jnp.broadcast_to(q0_ref[...], (C, R, W))


def _g1_sc_body(c1_ref, out_ref, buf_ref, sem, *, B1, C, M0, N1):
    n_workers = _NUM_SC_CORES * _NUM_SC_SUBCORES
    rows_per_w = M0 // n_workers
    wid = lax.axis_index("s") * _NUM_SC_CORES + lax.axis_index("c")
    batch = wid // (n_workers // B1)
    row0 = wid * rows_per_w
    for c in range(C):
        src_off = (batch * C + c) * N1
        loads = [
            pltpu.make_async_copy(
                c1_ref.at[pl.ds(src_off, N1)],
                buf_ref.at[pl.ds(k * N1, N1)],
                sem,
            )
            for k in range(_REP)
        ]
        for cp in loads:
            cp.start()
        for cp in loads:
            cp.wait()
        stores = [
            pltpu.make_async_copy(
                buf_ref,
                out_ref.at[pl.ds((c * M0 + row0 + t * _REP) * N1, _REP * N1)],
                sem,
            )
            for t in range(rows_per_w // _REP)
        ]
        for cp in stores:
            cp.start()
        for cp in stores:
            cp.wait()


def kernel(cloud0, cloud1):
    B0, C, N0 = cloud0.shape
    B1, _, N1 = cloud1.shape
    M0, M1 = B0 * N0, B1 * N1
    pts0 = jnp.transpose(cloud0, (0, 2, 1)).reshape(M0, C)
    pts1 = jnp.transpose(cloud1, (0, 2, 1)).reshape(M1, C)
    q0 = jnp.transpose(cloud0, (1, 0, 2)).reshape(C, M0, 1)

    R = 256
    g0t = pl.pallas_call(
        _g0_body,
        grid=(M0 // R,),
        in_specs=[pl.BlockSpec((C, R, 1), lambda r: (0, r, 0))],
        out_specs=pl.BlockSpec((C, R, N1), lambda r: (0, r, 0)),
        out_shape=jax.ShapeDtypeStruct((C, M0, N1), jnp.float32),
        compiler_params=pltpu.CompilerParams(
            dimension_semantics=("parallel",),
        ),
    )(q0)

    sc_kernel = functools.partial(
        pl.kernel,
        mesh=plsc.VectorSubcoreMesh(core_axis_name="c", subcore_axis_name="s"),
        out_type=jax.ShapeDtypeStruct((C * M0 * N1,), jnp.float32),
        scratch_types=[
            pltpu.VMEM((_REP * N1,), jnp.float32),
            pltpu.SemaphoreType.DMA,
        ],
    )(functools.partial(_g1_sc_body, B1=B1, C=C, M0=M0, N1=N1))
    g1t = sc_kernel(cloud1.reshape(-1)).reshape(C, M0, N1)

    return (
        pts0,
        pts1,
        jnp.transpose(g0t, (1, 2, 0)),
        jnp.transpose(g1t, (1, 2, 0)),
    )


# speedup vs baseline: 3.3236x; 3.3236x over previous
import jax
import jax.numpy as jnp
from jax.experimental import pallas as pl
from jax.experimental.pallas import tpu as pltpu


def _grouping_body(q0_ref, c1_ref, g0_ref, g1_ref):
    C, R, W = g0_ref.shape
    g0_ref[...] = jnp.broadcast_to(q0_ref[...], (C, R, W))
    g1_ref[...] = jnp.broadcast_to(c1_ref[0][:, None, :], (C, R, W))


def kernel(cloud0, cloud1):
    B0, C, N0 = cloud0.shape
    B1, _, N1 = cloud1.shape
    M0, M1 = B0 * N0, B1 * N1
    pts0 = jnp.transpose(cloud0, (0, 2, 1)).reshape(M0, C)
    pts1 = jnp.transpose(cloud1, (0, 2, 1)).reshape(M1, C)
    q0 = jnp.transpose(cloud0, (1, 0, 2)).reshape(C, M0, 1)

    R = 512
    grid = (M0 // R,)
    g0t, g1t = pl.pallas_call(
        _grouping_body,
        grid=grid,
        in_specs=[
            pl.BlockSpec((C, R, 1), lambda r: (0, r, 0)),
            pl.BlockSpec((1, C, N1), lambda r: (r * R // N0, 0, 0)),
        ],
        out_specs=[
            pl.BlockSpec((C, R, N1), lambda r: (0, r, 0)),
            pl.BlockSpec((C, R, N1), lambda r: (0, r, 0)),
        ],
        out_shape=[
            jax.ShapeDtypeStruct((C, M0, N1), jnp.float32),
            jax.ShapeDtypeStruct((C, M0, N1), jnp.float32),
        ],
        compiler_params=pltpu.CompilerParams(
            dimension_semantics=("parallel",),
        ),
    )(q0, cloud1)
    return (
        pts0,
        pts1,
        jnp.transpose(g0t, (1, 2, 0)),
        jnp.transpose(g1t, (1, 2, 0)),
    )


# speedup vs baseline: 3.3572x; 1.0101x over previous
import jax
import jax.numpy as jnp
from jax.experimental import pallas as pl
from jax.experimental.pallas import tpu as pltpu


def _grouping_body(q0_ref, c1_ref, g0_ref, g1_ref):
    C, R, W = g0_ref.shape
    g0_ref[...] = jnp.broadcast_to(q0_ref[...], (C, R, W))
    g1_ref[...] = jnp.broadcast_to(c1_ref[0][:, None, :], (C, R, W))


def kernel(cloud0, cloud1):
    B0, C, N0 = cloud0.shape
    B1, _, N1 = cloud1.shape
    M0, M1 = B0 * N0, B1 * N1
    pts0 = jnp.transpose(cloud0, (0, 2, 1)).reshape(M0, C)
    pts1 = jnp.transpose(cloud1, (0, 2, 1)).reshape(M1, C)
    q0 = jnp.transpose(cloud0, (1, 0, 2)).reshape(C, M0, 1)

    R = 256
    grid = (M0 // R,)
    g0t, g1t = pl.pallas_call(
        _grouping_body,
        grid=grid,
        in_specs=[
            pl.BlockSpec((C, R, 1), lambda r: (0, r, 0)),
            pl.BlockSpec((1, C, N1), lambda r: (r * R // N0, 0, 0)),
        ],
        out_specs=[
            pl.BlockSpec((C, R, N1), lambda r: (0, r, 0)),
            pl.BlockSpec((C, R, N1), lambda r: (0, r, 0)),
        ],
        out_shape=[
            jax.ShapeDtypeStruct((C, M0, N1), jnp.float32),
            jax.ShapeDtypeStruct((C, M0, N1), jnp.float32),
        ],
        compiler_params=pltpu.CompilerParams(
            dimension_semantics=("parallel",),
        ),
    )(q0, cloud1)
    return (
        pts0,
        pts1,
        jnp.transpose(g0t, (1, 2, 0)),
        jnp.transpose(g1t, (1, 2, 0)),
    )
